# Initial kernel scaffold; baseline (speedup 1.0000x reference)
import jax
import jax.numpy as jnp
from jax.experimental import pallas as pl


def kernel(x, grid_starts, grid_lengths, pair_starts, pair_lengths):
    raise NotImplementedError("write your pallas kernel here")


# speedup vs baseline: 7.4506x; 7.4506x over previous
import math

import jax
import jax.numpy as jnp
import numpy as np
from jax.experimental import pallas as pl
from jax.experimental.pallas import tpu as pltpu

D_MODEL = 512
MAX_PIXELS = 1024
BATCH = 4
SEQ = 4096
TS = 512
N_TILES = SEQ // TS
GP_ROWS = 32


def _pe_table(length, d_model):
    position = np.arange(length, dtype=np.float32)[:, None]
    div_term = np.exp(
        np.arange(0, d_model, 2, dtype=np.float32) * (-(math.log(10000.0) / d_model))
    )
    pe = np.zeros((length, d_model), dtype=np.float32)
    pe[:, 0::2] = np.sin(position * div_term)
    pe[:, 1::2] = np.cos(position * div_term)
    return pe


_PIX_TAB = _pe_table(MAX_PIXELS, D_MODEL)
_GRID_TAB = _pe_table(4, D_MODEL)
_PAIR_TAB = _pe_table(16, D_MODEL)

_GP_TAB = np.zeros((GP_ROWS, D_MODEL), dtype=np.float32)
for _g in (-1, 0, 1):
    for _p in range(-1, 8):
        _row = (_g + 1) * 9 + (_p + 1)
        if _g >= 0:
            _GP_TAB[_row] += _GRID_TAB[_g]
        if _p >= 0:
            _GP_TAB[_row] += _PAIR_TAB[_p]


def _tile_kernel(gs, gl, ps, pls, x_ref, pix_ref, gp_ref, out_ref):
    i = pl.program_id(0)
    tok = jax.lax.broadcasted_iota(jnp.int32, (TS, 1), 0) + i * TS

    pix_idx = jnp.full((TS, 1), MAX_PIXELS, jnp.int32)
    g = jnp.full((TS, 1), -1, jnp.int32)
    p = jnp.full((TS, 1), -1, jnp.int32)
    for k in range(16):
        s = gs[k]
        m = (tok >= s) & (tok < s + gl[k])
        pix_idx = jnp.where(m, jnp.minimum(tok - s, MAX_PIXELS - 1), pix_idx)
        g = jnp.where(m, k % 2, g)
    for k in range(16):
        s = ps[k]
        m = (tok >= s) & (tok < s + pls[k])
        p = jnp.where(m, k, p)
    code = (g + 1) * 9 + (p + 1)

    oh_pix = jnp.where(
        jax.lax.broadcasted_iota(jnp.int32, (TS, MAX_PIXELS), 1) == pix_idx, 1.0, 0.0
    )
    oh_gp = jnp.where(
        jax.lax.broadcasted_iota(jnp.int32, (TS, GP_ROWS), 1) == code, 1.0, 0.0
    )
    pe = jnp.dot(
        oh_pix, pix_ref[...], preferred_element_type=jnp.float32,
        precision=jax.lax.Precision.HIGHEST,
    ) + jnp.dot(
        oh_gp, gp_ref[...], preferred_element_type=jnp.float32,
        precision=jax.lax.Precision.HIGHEST,
    )
    out_ref[...] = x_ref[...] + pe[None, :, :]


def kernel(x, grid_starts, grid_lengths, pair_starts, pair_lengths):
    pad = jnp.zeros((8,), jnp.int32)
    ps16 = jnp.concatenate([pair_starts.astype(jnp.int32), pad])
    pl16 = jnp.concatenate([pair_lengths.astype(jnp.int32), pad])

    grid_spec = pltpu.PrefetchScalarGridSpec(
        num_scalar_prefetch=4,
        grid=(N_TILES,),
        in_specs=[
            pl.BlockSpec((BATCH, TS, D_MODEL), lambda i, *_: (0, i, 0)),
            pl.BlockSpec((MAX_PIXELS, D_MODEL), lambda i, *_: (0, 0)),
            pl.BlockSpec((GP_ROWS, D_MODEL), lambda i, *_: (0, 0)),
        ],
        out_specs=pl.BlockSpec((BATCH, TS, D_MODEL), lambda i, *_: (0, i, 0)),
    )
    return pl.pallas_call(
        _tile_kernel,
        grid_spec=grid_spec,
        out_shape=jax.ShapeDtypeStruct(x.shape, x.dtype),
    )(
        grid_starts.astype(jnp.int32),
        grid_lengths.astype(jnp.int32),
        ps16,
        pl16,
        x,
        jnp.asarray(_PIX_TAB),
        jnp.asarray(_GP_TAB),
    )


# speedup vs baseline: 12.9680x; 1.7405x over previous
import math

import jax
import jax.numpy as jnp
import numpy as np
from jax.experimental import pallas as pl
from jax.experimental.pallas import tpu as pltpu

D_MODEL = 512
MAX_PIXELS = 1024
BATCH = 4
SEQ = 4096
TS = 512
N_TILES = SEQ // TS
GP_ROWS = 32


def _pe_table(length, d_model):
    position = np.arange(length, dtype=np.float32)[:, None]
    div_term = np.exp(
        np.arange(0, d_model, 2, dtype=np.float32) * (-(math.log(10000.0) / d_model))
    )
    pe = np.zeros((length, d_model), dtype=np.float32)
    pe[:, 0::2] = np.sin(position * div_term)
    pe[:, 1::2] = np.cos(position * div_term)
    return pe


_PIX_TAB = _pe_table(MAX_PIXELS, D_MODEL)
_GRID_TAB = _pe_table(4, D_MODEL)
_PAIR_TAB = _pe_table(16, D_MODEL)

_GP_TAB = np.zeros((GP_ROWS, D_MODEL), dtype=np.float32)
for _g in (-1, 0, 1):
    for _p in range(-1, 8):
        _row = (_g + 1) * 9 + (_p + 1)
        if _g >= 0:
            _GP_TAB[_row] += _GRID_TAB[_g]
        if _p >= 0:
            _GP_TAB[_row] += _PAIR_TAB[_p]


def _tile_kernel(gs, gl, ps, pls, x_ref, pix_ref, gp_ref, out_ref):
    i = pl.program_id(0)
    tok = jax.lax.broadcasted_iota(jnp.int32, (TS, 1), 0) + i * TS

    pix_idx = jnp.full((TS, 1), MAX_PIXELS, jnp.int32)
    g = jnp.full((TS, 1), -1, jnp.int32)
    p = jnp.full((TS, 1), -1, jnp.int32)
    for k in range(16):
        s = gs[k]
        m = (tok >= s) & (tok < s + gl[k])
        pix_idx = jnp.where(m, jnp.minimum(tok - s, MAX_PIXELS - 1), pix_idx)
        g = jnp.where(m, k % 2, g)
    for k in range(16):
        s = ps[k]
        m = (tok >= s) & (tok < s + pls[k])
        p = jnp.where(m, k, p)
    code = (g + 1) * 9 + (p + 1)

    oh_pix = jnp.where(
        jax.lax.broadcasted_iota(jnp.int32, (TS, MAX_PIXELS), 1) == pix_idx, 1.0, 0.0
    )
    oh_gp = jnp.where(
        jax.lax.broadcasted_iota(jnp.int32, (TS, GP_ROWS), 1) == code, 1.0, 0.0
    )
    pe = jnp.dot(
        oh_pix, pix_ref[...], preferred_element_type=jnp.float32,
        precision=jax.lax.Precision.DEFAULT,
    ) + jnp.dot(
        oh_gp, gp_ref[...], preferred_element_type=jnp.float32,
        precision=jax.lax.Precision.DEFAULT,
    )
    out_ref[...] = x_ref[...] + pe[None, :, :]


def kernel(x, grid_starts, grid_lengths, pair_starts, pair_lengths):
    pad = jnp.zeros((8,), jnp.int32)
    ps16 = jnp.concatenate([pair_starts.astype(jnp.int32), pad])
    pl16 = jnp.concatenate([pair_lengths.astype(jnp.int32), pad])

    grid_spec = pltpu.PrefetchScalarGridSpec(
        num_scalar_prefetch=4,
        grid=(N_TILES,),
        in_specs=[
            pl.BlockSpec((BATCH, TS, D_MODEL), lambda i, *_: (0, i, 0)),
            pl.BlockSpec((MAX_PIXELS, D_MODEL), lambda i, *_: (0, 0)),
            pl.BlockSpec((GP_ROWS, D_MODEL), lambda i, *_: (0, 0)),
        ],
        out_specs=pl.BlockSpec((BATCH, TS, D_MODEL), lambda i, *_: (0, i, 0)),
    )
    return pl.pallas_call(
        _tile_kernel,
        grid_spec=grid_spec,
        out_shape=jax.ShapeDtypeStruct(x.shape, x.dtype),
    )(
        grid_starts.astype(jnp.int32),
        grid_lengths.astype(jnp.int32),
        ps16,
        pl16,
        x,
        jnp.asarray(_PIX_TAB),
        jnp.asarray(_GP_TAB),
    )
